# Initial kernel scaffold; baseline (speedup 1.0000x reference)
import jax
import jax.numpy as jnp
from jax.experimental import pallas as pl


def kernel(query_rel, temp_neighbors_facts, tail_nodes, tail_index, hidden, ent_emds, rel_emds, time_encoder, line_time, rel_emd_table, Ws_W, Wr_W, Wqr_W, Wqr_b, walpha_W, walpha_b, Wh_W):
    raise NotImplementedError("write your pallas kernel here")


# speedup vs baseline: 1.9750x; 1.9750x over previous
import functools

import jax
import jax.numpy as jnp
from jax import lax
from jax.experimental import pallas as pl
from jax.experimental.pallas import tpu as pltpu
from jax.experimental.pallas import tpu_sc as plsc

_N = 10000
_E = 320000
_D = 128
_NRE = 10001
_NRE_PAD = 10240

_NC = 2
_NS = 16
_NW = _NC * _NS
_EPT = _E // _NW
_K = 40
_NCHUNK = _EPT // _K
_RPT = _N // _NS

_DOT = functools.partial(
    lax.dot_general,
    dimension_numbers=(((1,), (1,)), ((), ())),
    preferred_element_type=jnp.float32,
    precision=lax.Precision.HIGHEST,
)


def _hidden_proj_body(x_ref, w_ref, o_ref):
    x = x_ref[...]
    o_ref[...] = jnp.concatenate([_DOT(x, w_ref[...]), x], axis=1)


def _rel_proj_body(x_ref, wr_ref, wq_ref, bq_ref, b2_ref, bq_out_ref):
    x = x_ref[...]
    b2_ref[...] = jnp.concatenate([_DOT(x, wr_ref[...]), x], axis=1)
    bq_out_ref[...] = _DOT(x, wq_ref[...]) + bq_ref[...]


def _final_body(p_ref, w_ref, o_ref):
    o_ref[...] = _DOT(p_ref[0] + p_ref[1], w_ref[...])


def _sc_edge_body(a2_hbm, b2_hbm, bq_hbm, n_hbm, r_hbm, b_hbm, t_hbm,
                  q_hbm, wal_hbm, wb_hbm, out_hbm,
                  ha, hb, hc, msg, nb, rb, bb, tb, cb, walv, wbv, red,
                  agg, sem, sem_q):
    c = lax.axis_index("c")
    s = lax.axis_index("s")
    wid = c * _NS + s

    pltpu.sync_copy(wal_hbm, walv)
    pltpu.sync_copy(wb_hbm, wbv)

    def _zero(e, carry):
        for j in range(8):
            msg[e, pl.ds(16 * j, 16)] = jnp.zeros((16,), jnp.float32)
        return carry
    lax.fori_loop(0, _K, _zero, 0)
    red[pl.ds(16, 16)] = jnp.zeros((16,), jnp.float32)
    zbase = pl.multiple_of(s * 624, 8)
    for off in tuple(range(0, 624 - _K + 1, _K)) + (624 - _K,):
        pltpu.sync_copy(msg, agg.at[pl.ds(zbase + off, _K)])

    @pl.when(s == _NS - 1)
    def _zero_tail():
        pltpu.sync_copy(msg, agg.at[pl.ds(zbase + 640 - _K, _K)])

    plsc.subcore_barrier()

    ebase = wid * _EPT

    def _chunk(ci, carry):
        off = pl.multiple_of(ebase + ci * _K, 8)
        pltpu.sync_copy(n_hbm.at[pl.ds(off, _K)], nb)
        pltpu.sync_copy(r_hbm.at[pl.ds(off, _K)], rb)
        pltpu.sync_copy(b_hbm.at[pl.ds(off, _K)], bb)
        pltpu.sync_copy(t_hbm.at[pl.ds(off, _K)], tb)
        cp_q = pltpu.async_copy(q_hbm.at[bb], cb, sem_q)
        cp_a = pltpu.async_copy(a2_hbm.at[nb], ha, sem)
        cp_b = pltpu.async_copy(b2_hbm.at[rb], hb, sem)
        cp_q.wait()
        cp_c = pltpu.async_copy(bq_hbm.at[cb], hc, sem)
        cp_a.wait()
        cp_b.wait()
        cp_c.wait()

        wal = [walv[j] for j in range(8)]
        wb = wbv[...]

        def _edge(e, ecarry):
            acc = jnp.zeros((16,), jnp.float32)
            for j in range(8):
                pre = jnp.maximum(ha[e, pl.ds(16 * j, 16)]
                                  + hb[e, pl.ds(16 * j, 16)]
                                  + hc[e, pl.ds(16 * j, 16)], 0.0)
                acc = acc + pre * wal[j]
            red[pl.ds(0, 16)] = acc
            t = acc + red[pl.ds(8, 16)]
            ssum = t[0]
            for j in range(1, 8):
                ssum = ssum + t[j]
            sv = jnp.full((16,), ssum, jnp.float32) + wb
            alpha = 1.0 / (1.0 + jnp.exp(-sv))
            for j in range(8):
                msg[e, pl.ds(16 * j, 16)] = (
                    ha[e, pl.ds(128 + 16 * j, 16)]
                    + hb[e, pl.ds(128 + 16 * j, 16)]) * alpha
            return ecarry

        lax.fori_loop(0, _K, _edge, 0)
        pltpu.sync_copy(msg, agg.at[tb], add=True)
        return carry

    lax.fori_loop(0, _NCHUNK, _chunk, 0)

    plsc.subcore_barrier()
    rbase = pl.multiple_of(s * 624, 8)

    @pl.when(s < _NS - 1)
    def _write_main():
        pltpu.sync_copy(agg.at[pl.ds(rbase, 624)],
                        out_hbm.at[c, pl.ds(rbase, 624)])

    @pl.when(s == _NS - 1)
    def _write_tail():
        pltpu.sync_copy(agg.at[pl.ds(rbase, 640)],
                        out_hbm.at[c, pl.ds(rbase, 640)])


def kernel(query_rel, temp_neighbors_facts, tail_nodes, tail_index, hidden,
           ent_emds, rel_emds, time_encoder, line_time,
           rel_emd_table, Ws_W, Wr_W, Wqr_W, Wqr_b, walpha_W, walpha_b, Wh_W):
    del tail_nodes, ent_emds, rel_emds, time_encoder, line_time

    n_idx = temp_neighbors_facts[:, 1]
    r_idx = temp_neighbors_facts[:, 3] + 1
    b_idx = temp_neighbors_facts[:, 0]
    qidx = query_rel + 1

    rel_pad = jnp.pad(rel_emd_table, ((0, _NRE_PAD - _NRE), (0, 0)))

    a2 = pl.pallas_call(
        _hidden_proj_body,
        grid=(25,),
        in_specs=[
            pl.BlockSpec((400, _D), lambda i: (i, 0)),
            pl.BlockSpec((_D, _D), lambda i: (0, 0)),
        ],
        out_specs=pl.BlockSpec((400, 2 * _D), lambda i: (i, 0)),
        out_shape=jax.ShapeDtypeStruct((_N, 2 * _D), jnp.float32),
    )(hidden, Ws_W)

    b2, bq = pl.pallas_call(
        _rel_proj_body,
        grid=(20,),
        in_specs=[
            pl.BlockSpec((512, _D), lambda i: (i, 0)),
            pl.BlockSpec((_D, _D), lambda i: (0, 0)),
            pl.BlockSpec((_D, _D), lambda i: (0, 0)),
            pl.BlockSpec((1, _D), lambda i: (0, 0)),
        ],
        out_specs=[
            pl.BlockSpec((512, 2 * _D), lambda i: (i, 0)),
            pl.BlockSpec((512, _D), lambda i: (i, 0)),
        ],
        out_shape=[
            jax.ShapeDtypeStruct((_NRE_PAD, 2 * _D), jnp.float32),
            jax.ShapeDtypeStruct((_NRE_PAD, _D), jnp.float32),
        ],
    )(rel_pad, Wr_W, Wqr_W, Wqr_b.reshape(1, _D))

    mesh = plsc.VectorSubcoreMesh(core_axis_name="c", subcore_axis_name="s")
    edge_fn = pl.kernel(
        _sc_edge_body,
        mesh=mesh,
        out_type=jax.ShapeDtypeStruct((_NC, _N, _D), jnp.float32),
        scratch_types=[
            pltpu.VMEM((_K, 2 * _D), jnp.float32),
            pltpu.VMEM((_K, 2 * _D), jnp.float32),
            pltpu.VMEM((_K, _D), jnp.float32),
            pltpu.VMEM((_K, _D), jnp.float32),
            pltpu.VMEM((_K,), jnp.int32),
            pltpu.VMEM((_K,), jnp.int32),
            pltpu.VMEM((_K,), jnp.int32),
            pltpu.VMEM((_K,), jnp.int32),
            pltpu.VMEM((_K,), jnp.int32),
            pltpu.VMEM((8, 16), jnp.float32),
            pltpu.VMEM((16,), jnp.float32),
            pltpu.VMEM((32,), jnp.float32),
            pltpu.VMEM_SHARED((_N, _D), jnp.float32),
            pltpu.SemaphoreType.DMA,
            pltpu.SemaphoreType.DMA,
        ],
    )

    partials = edge_fn(
        a2, b2, bq,
        n_idx, r_idx, b_idx, tail_index, qidx,
        walpha_W.reshape(8, 16),
        jnp.full((16,), walpha_b[0], jnp.float32),
    )

    out = pl.pallas_call(
        _final_body,
        grid=(25,),
        in_specs=[
            pl.BlockSpec((_NC, 400, _D), lambda i: (0, i, 0)),
            pl.BlockSpec((_D, _D), lambda i: (0, 0)),
        ],
        out_specs=pl.BlockSpec((400, _D), lambda i: (i, 0)),
        out_shape=jax.ShapeDtypeStruct((_N, _D), jnp.float32),
    )(partials, Wh_W)

    return out
